# baseline (speedup 1.0000x reference)
import functools

import jax
import jax.numpy as jnp
from jax import lax
from jax.experimental import pallas as pl
from jax.experimental.pallas import tpu as pltpu
from jax.experimental.pallas import tpu_sc as plsc

_D = 1024
_WDIM = 20
_WPAD = 32
_VOCAB = 8
_BS = 1024
_L = 16


def _sc_gather(rows):
    info = plsc.get_sparse_core_info()
    nw = info.num_cores * info.num_subcores
    b_per_w = rows // nw
    mesh = plsc.VectorSubcoreMesh(core_axis_name="c", subcore_axis_name="s")

    @functools.partial(
        pl.kernel,
        mesh=mesh,
        compiler_params=pltpu.CompilerParams(use_tc_tiling_on_sc=False),
        out_type=jax.ShapeDtypeStruct((rows, _WPAD), jnp.float32),
        scratch_types=[
            pltpu.VMEM((b_per_w,), jnp.int32),
            pltpu.VMEM((b_per_w,), jnp.int32),
            pltpu.VMEM((b_per_w,), jnp.int32),
            pltpu.VMEM((b_per_w, _WPAD), jnp.float32),
            pltpu.SemaphoreType.DMA,
        ],
    )
    def k(starts_hbm, ends_hbm, table_hbm, wemb_hbm,
          starts_v, ends_v, idx_v, rows_v, sem):
        wid = lax.axis_index("s") * info.num_cores + lax.axis_index("c")
        base = wid * b_per_w
        pltpu.sync_copy(starts_hbm.at[pl.ds(base, b_per_w)], starts_v)
        pltpu.sync_copy(ends_hbm.at[pl.ds(base, b_per_w)], ends_v)

        def body(i, _):
            sl = pl.ds(i * _L, _L)
            idx_v[sl] = ends_v[sl] - starts_v[sl]
            return 0

        lax.fori_loop(0, b_per_w // _L, body, 0, unroll=4)
        pltpu.async_copy(table_hbm.at[idx_v], rows_v, sem).wait()
        pltpu.sync_copy(rows_v, wemb_hbm.at[pl.ds(base, b_per_w)])

    return k


def _tc_body(emb_ref, w_ref, out_ref):
    out_ref[:, :_D] = emb_ref[...]
    out_ref[:, _D:] = w_ref[:, :_WDIM]


def kernel(spans, span_embeddings, width_table):
    B, S, D = span_embeddings.shape
    rows = B * S
    nb = rows // _BS
    starts = spans[..., 0].astype(jnp.int32).reshape(rows)
    ends = spans[..., 1].astype(jnp.int32).reshape(rows)
    table_pad = jnp.zeros((_VOCAB, _WPAD), jnp.float32).at[:, :_WDIM].set(width_table)
    emb = span_embeddings.reshape(rows, D)

    wemb = _sc_gather(rows)(starts, ends, table_pad)

    out = pl.pallas_call(
        _tc_body,
        grid=(nb,),
        in_specs=[
            pl.BlockSpec((_BS, D), lambda i: (i, 0)),
            pl.BlockSpec((_BS, _WPAD), lambda i: (i, 0)),
        ],
        out_specs=pl.BlockSpec((_BS, D + _WDIM), lambda i: (i, 0)),
        out_shape=jax.ShapeDtypeStruct((rows, D + _WDIM), jnp.float32),
    )(emb, wemb)
    return out.reshape(B, S, D + _WDIM)


# speedup vs baseline: 2.3013x; 2.3013x over previous
import functools

import jax
import jax.numpy as jnp
from jax import lax
from jax.experimental import pallas as pl
from jax.experimental.pallas import tpu as pltpu
from jax.experimental.pallas import tpu_sc as plsc

_D = 1024
_WDIM = 20
_WPAD = 32
_VOCAB = 8
_BS = 1024
_L = 16


def _sc_gather(rows):
    info = plsc.get_sparse_core_info()
    nw = info.num_cores * info.num_subcores
    b_per_w = rows // nw
    mesh = plsc.VectorSubcoreMesh(core_axis_name="c", subcore_axis_name="s")

    chunk = 512

    @functools.partial(
        pl.kernel,
        mesh=mesh,
        compiler_params=pltpu.CompilerParams(needs_layout_passes=False),
        out_type=jax.ShapeDtypeStruct((rows * 128,), jnp.float32),
        scratch_types=[
            pltpu.VMEM((b_per_w,), jnp.int32),
            pltpu.VMEM((b_per_w,), jnp.int32),
            pltpu.VMEM((_VOCAB * _WPAD,), jnp.float32),
            pltpu.VMEM((chunk * 128,), jnp.float32),
        ],
    )
    def k(starts_hbm, ends_hbm, table_hbm, wemb_hbm,
          starts_v, ends_v, table_v, rows_f):
        wid = lax.axis_index("s") * info.num_cores + lax.axis_index("c")
        base = wid * b_per_w
        pltpu.sync_copy(starts_hbm.at[pl.ds(base, b_per_w)], starts_v)
        pltpu.sync_copy(ends_hbm.at[pl.ds(base, b_per_w)], ends_v)
        pltpu.sync_copy(table_hbm, table_v)
        lane = lax.iota(jnp.int32, _L)

        for h in range(b_per_w // chunk):
            def body(g, _):
                sl = pl.ds(h * chunk + g * _L, _L)
                w_vec = ends_v[sl] - starts_v[sl]
                tbase = w_vec * _WPAD
                rbase = (g * _L + lane) * 128
                for c in range(_WDIM):
                    val = plsc.load_gather(table_v, [tbase + c])
                    plsc.store_scatter(rows_f, [rbase + c], val)
                return 0

            lax.fori_loop(0, chunk // _L, body, 0, unroll=1)
            pltpu.sync_copy(
                rows_f,
                wemb_hbm.at[pl.ds((base + h * chunk) * 128, chunk * 128)])

    return k


def _tc_body(emb_ref, w_ref, out_ref):
    out_ref[:, :_D] = emb_ref[...]
    out_ref[:, _D:] = w_ref[:, :_WDIM]


def kernel(spans, span_embeddings, width_table):
    B, S, D = span_embeddings.shape
    rows = B * S
    nb = rows // _BS
    starts = spans[..., 0].astype(jnp.int32).reshape(rows)
    ends = spans[..., 1].astype(jnp.int32).reshape(rows)
    table_pad = jnp.zeros((_VOCAB, _WPAD), jnp.float32).at[:, :_WDIM].set(width_table)
    emb = span_embeddings.reshape(rows, D)

    wemb = _sc_gather(rows)(starts, ends, table_pad.reshape(-1)).reshape(rows, 128)

    out = pl.pallas_call(
        _tc_body,
        grid=(nb,),
        in_specs=[
            pl.BlockSpec((_BS, D), lambda i: (i, 0)),
            pl.BlockSpec((_BS, 128), lambda i: (i, 0)),
        ],
        out_specs=pl.BlockSpec((_BS, D + _WDIM), lambda i: (i, 0)),
        out_shape=jax.ShapeDtypeStruct((rows, D + _WDIM), jnp.float32),
    )(emb, wemb)
    return out.reshape(B, S, D + _WDIM)


# speedup vs baseline: 2.3211x; 1.0086x over previous
import functools

import jax
import jax.numpy as jnp
from jax import lax
from jax.experimental import pallas as pl
from jax.experimental.pallas import tpu as pltpu
from jax.experimental.pallas import tpu_sc as plsc

_D = 1024
_WDIM = 20
_WPAD = 32
_VOCAB = 8
_BS = 2048
_L = 16


def _sc_gather(rows):
    info = plsc.get_sparse_core_info()
    nw = info.num_cores * info.num_subcores
    b_per_w = rows // nw
    mesh = plsc.VectorSubcoreMesh(core_axis_name="c", subcore_axis_name="s")

    chunk = 512

    @functools.partial(
        pl.kernel,
        mesh=mesh,
        compiler_params=pltpu.CompilerParams(needs_layout_passes=False),
        out_type=jax.ShapeDtypeStruct((rows * 128,), jnp.float32),
        scratch_types=[
            pltpu.VMEM((b_per_w,), jnp.int32),
            pltpu.VMEM((b_per_w,), jnp.int32),
            pltpu.VMEM((_VOCAB * _WPAD,), jnp.float32),
            pltpu.VMEM((chunk * 128,), jnp.float32),
        ],
    )
    def k(starts_hbm, ends_hbm, table_hbm, wemb_hbm,
          starts_v, ends_v, table_v, rows_f):
        wid = lax.axis_index("s") * info.num_cores + lax.axis_index("c")
        base = wid * b_per_w
        pltpu.sync_copy(starts_hbm.at[pl.ds(base, b_per_w)], starts_v)
        pltpu.sync_copy(ends_hbm.at[pl.ds(base, b_per_w)], ends_v)
        pltpu.sync_copy(table_hbm, table_v)
        lane = lax.iota(jnp.int32, _L)

        for h in range(b_per_w // chunk):
            def body(g, _):
                sl = pl.ds(h * chunk + g * _L, _L)
                w_vec = ends_v[sl] - starts_v[sl]
                tbase = w_vec * _WPAD
                rbase = (g * _L + lane) * 128
                for c in range(_WDIM):
                    val = plsc.load_gather(table_v, [tbase + c])
                    plsc.store_scatter(rows_f, [rbase + c], val)
                return 0

            lax.fori_loop(0, chunk // _L, body, 0, unroll=1)
            pltpu.sync_copy(
                rows_f,
                wemb_hbm.at[pl.ds((base + h * chunk) * 128, chunk * 128)])

    return k


def _tc_body(emb_ref, w_ref, out_ref):
    out_ref[:, :_D] = emb_ref[...]
    out_ref[:, _D:] = w_ref[:, :_WDIM]


def kernel(spans, span_embeddings, width_table):
    B, S, D = span_embeddings.shape
    rows = B * S
    nb = rows // _BS
    starts = spans[..., 0].astype(jnp.int32).reshape(rows)
    ends = spans[..., 1].astype(jnp.int32).reshape(rows)
    table_pad = jnp.zeros((_VOCAB, _WPAD), jnp.float32).at[:, :_WDIM].set(width_table)
    emb = span_embeddings.reshape(rows, D)

    wemb = _sc_gather(rows)(starts, ends, table_pad.reshape(-1)).reshape(rows, 128)

    out = pl.pallas_call(
        _tc_body,
        grid=(nb,),
        in_specs=[
            pl.BlockSpec((_BS, D), lambda i: (i, 0)),
            pl.BlockSpec((_BS, 128), lambda i: (i, 0)),
        ],
        out_specs=pl.BlockSpec((_BS, D + _WDIM), lambda i: (i, 0)),
        out_shape=jax.ShapeDtypeStruct((rows, D + _WDIM), jnp.float32),
    )(emb, wemb)
    return out.reshape(B, S, D + _WDIM)
